# Initial kernel scaffold; baseline (speedup 1.0000x reference)
import jax
import jax.numpy as jnp
from jax.experimental import pallas as pl


def kernel(x, W1, b1, W2, b2, Wg, bg):
    raise NotImplementedError("write your pallas kernel here")


# speedup vs baseline: 2.9278x; 2.9278x over previous
import functools

import jax
import jax.numpy as jnp
from jax.experimental import pallas as pl
from jax.experimental.pallas import tpu as pltpu

HIDDEN = 768
MLP_DIM = 3072
NUM_EXPERTS = 8
TOKENS = 2048
CHUNK = 256
SQRT_HALF = 0.7071067811865476


def _router_kernel(x_ref, wg_ref, bg_ref, gated_ref, eid_ref, wts_ref,
                   imp_ref):
    x = x_ref[...].astype(jnp.bfloat16)
    logits = jax.lax.dot_general(
        x, wg_ref[...].astype(jnp.bfloat16), (((1,), (0,)), ((), ())),
        preferred_element_type=jnp.float32) + bg_ref[...]
    m = jnp.max(logits, axis=-1, keepdims=True)
    ex = jnp.exp(logits - m)
    gates = ex / jnp.sum(ex, axis=-1, keepdims=True)

    imp = jnp.sum(gates, axis=0)
    imp_mean = jnp.mean(imp)
    imp_var = jnp.mean((imp - imp_mean) ** 2)
    imp_ref[...] = (imp_var / (imp_mean + 1e-9) ** 2).reshape(1, 1)

    lane = jax.lax.broadcasted_iota(jnp.int32, gates.shape, 1)
    m1 = jnp.max(gates, axis=-1, keepdims=True)
    e1 = jnp.min(jnp.where(gates >= m1, lane, NUM_EXPERTS), axis=-1,
                 keepdims=True)
    masked = jnp.where(lane == e1, -jnp.inf, gates)
    m2 = jnp.max(masked, axis=-1, keepdims=True)
    e2 = jnp.min(jnp.where(masked >= m2, lane, NUM_EXPERTS), axis=-1,
                 keepdims=True)

    mask = (gates >= m2).astype(jnp.float32)
    gsel = gates * mask
    denom = jnp.sum(gsel, axis=-1, keepdims=True) + 1e-9
    gated_ref[...] = gsel / denom

    eid_ref[...] = jnp.concatenate([e1, e2], axis=1)
    w1 = m1 / denom
    w2 = m2 / denom
    wts_ref[...] = jnp.concatenate([w1, w2], axis=1)


def _router(x2, Wg, bg):
    return pl.pallas_call(
        _router_kernel,
        out_shape=(
            jax.ShapeDtypeStruct((TOKENS, NUM_EXPERTS), jnp.float32),
            jax.ShapeDtypeStruct((TOKENS, 2), jnp.int32),
            jax.ShapeDtypeStruct((TOKENS, 2), jnp.float32),
            jax.ShapeDtypeStruct((1, 1), jnp.float32),
        ),
    )(x2, Wg, bg.reshape(1, NUM_EXPERTS))


def _dense_moe_kernel(x_ref, w1_ref, b1_ref, w2_ref, b2_ref, gated_ref,
                      out_ref, acc_ref):
    j = pl.program_id(1)

    @pl.when(j == 0)
    def _():
        acc_ref[...] = jnp.zeros_like(acc_ref)

    xb = x_ref[...].astype(jnp.bfloat16)
    h = jax.lax.dot_general(
        xb, w1_ref[0], (((1,), (0,)), ((), ())),
        preferred_element_type=jnp.float32) + b1_ref[0]
    h = h * 0.5 * (1.0 + jax.lax.erf(h * SQRT_HALF))
    eo = jax.lax.dot_general(
        h.astype(jnp.bfloat16), w2_ref[0], (((1,), (0,)), ((), ())),
        preferred_element_type=jnp.float32) + b2_ref[0]

    g = gated_ref[...]
    lane = jax.lax.broadcasted_iota(jnp.int32, g.shape, 1)
    gcol = jnp.sum(jnp.where(lane == j, g, 0.0), axis=-1, keepdims=True)
    acc_ref[...] += gcol * eo

    @pl.when(j == NUM_EXPERTS - 1)
    def _():
        out_ref[...] = acc_ref[...]


def _dense_moe(x2, W1b, b1, W2b, b2, gated):
    nchunks = TOKENS // CHUNK
    return pl.pallas_call(
        _dense_moe_kernel,
        grid=(nchunks, NUM_EXPERTS),
        in_specs=[
            pl.BlockSpec((CHUNK, HIDDEN), lambda i, j: (i, 0)),
            pl.BlockSpec((1, HIDDEN, MLP_DIM), lambda i, j: (j, 0, 0)),
            pl.BlockSpec((1, 1, MLP_DIM), lambda i, j: (j, 0, 0)),
            pl.BlockSpec((1, MLP_DIM, HIDDEN), lambda i, j: (j, 0, 0)),
            pl.BlockSpec((1, 1, HIDDEN), lambda i, j: (j, 0, 0)),
            pl.BlockSpec((CHUNK, NUM_EXPERTS), lambda i, j: (i, 0)),
        ],
        out_specs=pl.BlockSpec((CHUNK, HIDDEN), lambda i, j: (i, 0)),
        out_shape=jax.ShapeDtypeStruct((TOKENS, HIDDEN), jnp.float32),
        scratch_shapes=[pltpu.VMEM((CHUNK, HIDDEN), jnp.float32)],
    )(x2, W1b, b1.reshape(NUM_EXPERTS, 1, MLP_DIM), W2b,
      b2.reshape(NUM_EXPERTS, 1, HIDDEN), gated)


@jax.jit
def kernel(x, W1, b1, W2, b2, Wg, bg):
    b, s, h = x.shape
    x2 = x.reshape(b * s, h)
    gated, eid, wts, imp = _router(x2, Wg, bg)
    out = _dense_moe(x2, W1.astype(jnp.bfloat16), b1,
                     W2.astype(jnp.bfloat16), b2, gated)
    return out.reshape(b, s, h), imp[0, 0]
